# baseline (speedup 1.0000x reference)
import jax
import jax.numpy as jnp
from jax import lax
from jax.experimental import pallas as pl
from jax.experimental.pallas import tpu as pltpu

N = 4096
D = 256
BM = 256
I = N // BM
B = 512
NB = N // B


def _body(x_ref, adj_ref, w1_ref, wn_ref, b_ref, out_ref,
          adjbf_ref, xwa_ref, xwb_ref):
    p = pl.program_id(0)
    i = pl.program_id(1)

    @pl.when((p == 0) & (i == 0))
    def _():
        xwb_ref[...] = jnp.dot(
            x_ref[...], w1_ref[...], preferred_element_type=jnp.float32
        ).astype(jnp.bfloat16)

    @pl.when(p == 0)
    def _():
        r = pl.ds(i * BM, BM)
        ab = adj_ref[...].astype(jnp.bfloat16)
        adjbf_ref[r, :] = ab
        acc = jnp.dot(ab, xwb_ref[...], preferred_element_type=jnp.float32)
        h = jnp.maximum(acc + b_ref[0], 0.0).astype(jnp.bfloat16)
        xwa_ref[r, :] = jnp.dot(
            h, wn_ref[0], preferred_element_type=jnp.float32
        ).astype(jnp.bfloat16)

    @pl.when((p == 1) & (i == 0))
    def _():
        def l1_chunk(j, carry):
            r = pl.ds(j * B, B)
            acc = jnp.dot(
                adjbf_ref[r, :], xwa_ref[...],
                preferred_element_type=jnp.float32,
            )
            h = jnp.maximum(acc + b_ref[1], 0.0).astype(jnp.bfloat16)
            xwb_ref[r, :] = jnp.dot(
                h, wn_ref[1], preferred_element_type=jnp.float32
            ).astype(jnp.bfloat16)
            return carry

        lax.fori_loop(0, NB, l1_chunk, 0)

    @pl.when((p == 2) & (i == 0))
    def _():
        def l2_chunk(j, carry):
            r = pl.ds(j * B, B)
            acc = jnp.dot(
                adjbf_ref[r, :], xwb_ref[...],
                preferred_element_type=jnp.float32,
            )
            out_ref[r, :] = jnp.maximum(acc + b_ref[2], 0.0)
            return carry

        lax.fori_loop(0, NB, l2_chunk, 0)


@jax.jit
def kernel(x, adj, W1, b1, W2, b2, W3, b3):
    xbf = x.astype(jnp.bfloat16)
    w1 = W1.astype(jnp.bfloat16)
    wn = jnp.stack([W2, W3]).astype(jnp.bfloat16)
    b = jnp.stack([b1, b2, b3]).reshape(3, 1, D)

    last = I - 1
    return pl.pallas_call(
        _body,
        grid=(3, I),
        in_specs=[
            pl.BlockSpec((N, D), lambda p, i: (0, 0)),
            pl.BlockSpec((BM, N), lambda p, i: (jnp.where(p == 0, i, last), 0)),
            pl.BlockSpec((D, D), lambda p, i: (0, 0)),
            pl.BlockSpec((2, D, D), lambda p, i: (0, 0, 0)),
            pl.BlockSpec((3, 1, D), lambda p, i: (0, 0, 0)),
        ],
        out_specs=pl.BlockSpec((N, D), lambda p, i: (0, 0)),
        out_shape=jax.ShapeDtypeStruct((N, D), jnp.float32),
        scratch_shapes=[
            pltpu.VMEM((N, N), jnp.bfloat16),
            pltpu.VMEM((N, D), jnp.bfloat16),
            pltpu.VMEM((N, D), jnp.bfloat16),
        ],
        compiler_params=pltpu.CompilerParams(
            dimension_semantics=("arbitrary", "arbitrary"),
        ),
    )(xbf, adj, w1, wn, b)
